# baseline (speedup 1.0000x reference)
import functools
import jax
import jax.numpy as jnp
from jax import lax
from jax.experimental import pallas as pl
from jax.experimental.pallas import tpu as pltpu
from jax.experimental.pallas import tpu_sc as plsc

N = 10000
NP = 10240
D = 256
QD = 64
E = 160000
TN = 1000
GN = N // TN

K = 80
NBUF = 10
MAIN_CHUNKS = 120
EC_T = E // 16
CHUNKS_T = EC_T // K

ET = E // 32
NV = ET // 16
STRIPE = NP // 16

_mesh = plsc.VectorSubcoreMesh(core_axis_name="c", subcore_axis_name="s")
_sc_params = pltpu.CompilerParams(needs_layout_passes=False)
_sc_agg_params = pltpu.CompilerParams(
    needs_layout_passes=False, use_tc_tiling_on_sc=False)


@functools.partial(
    pl.kernel, mesh=_mesh, compiler_params=_sc_params,
    out_type=jax.ShapeDtypeStruct((2, NP), jnp.float32),
    scratch_types=[
        pltpu.VMEM((ET,), jnp.int32),
        pltpu.VMEM((NP,), jnp.float32),
        pltpu.VMEM((STRIPE,), jnp.float32),
        pltpu.VMEM((STRIPE,), jnp.float32),
        pltpu.VMEM_SHARED((16, NP), jnp.float32),
    ])
def _deg_kernel(dst_hbm, deg_out, dst_v, acc_v, red_v, tmp_v, part_sh):
    c = lax.axis_index("c")
    s = lax.axis_index("s")
    tid = c * 16 + s
    pltpu.sync_copy(dst_hbm.at[pl.ds(tid * ET, ET)], dst_v)

    zero16 = jnp.zeros((16,), jnp.float32)
    ones16 = jnp.ones((16,), jnp.float32)

    def zbody(i, _):
        acc_v[pl.ds(i * 16, 16)] = zero16
        return 0
    lax.fori_loop(0, NP // 16, zbody, 0)

    def body(i, _):
        idx = dst_v[pl.ds(i * 16, 16)]
        plsc.addupdate_scatter(acc_v, [idx], ones16)
        return 0
    lax.fori_loop(0, NV, body, 0)
    rem_idx = dst_v[pl.ds(ET - 16, 16)]
    rem_mask = lax.iota(jnp.int32, 16) >= (16 - (ET - NV * 16))
    plsc.addupdate_scatter(acc_v, [rem_idx], ones16, mask=rem_mask)

    pltpu.sync_copy(acc_v, part_sh.at[s])
    plsc.subcore_barrier()

    sl = pl.ds(s * STRIPE, STRIPE)
    pltpu.sync_copy(part_sh.at[0, sl], red_v)

    def rbody(j, _):
        pltpu.sync_copy(part_sh.at[j, sl], tmp_v)

        def abody(i, _):
            w = pl.ds(i * 16, 16)
            red_v[w] = red_v[w] + tmp_v[w]
            return 0
        lax.fori_loop(0, STRIPE // 16, abody, 0)
        return 0
    lax.fori_loop(1, 16, rbody, 0)
    pltpu.sync_copy(red_v, deg_out.at[c, sl])


@functools.partial(
    pl.kernel, mesh=_mesh, compiler_params=_sc_agg_params,
    out_type=jax.ShapeDtypeStruct((2 * NP, QD), jnp.float32),
    scratch_types=[
        pltpu.VMEM((CHUNKS_T, K), jnp.int32),
        pltpu.VMEM((CHUNKS_T, K), jnp.int32),
    ] + [pltpu.VMEM((K, QD), jnp.float32) for _ in range(NBUF)]
      + [pltpu.SemaphoreType.DMA for _ in range(NBUF)]
      + [pltpu.VMEM_SHARED((NP, QD), jnp.float32)])
def _agg_kernel(hs_hbm, srcs2_hbm, dst2_hbm, out_hbm, src_t, dst_t,
                r0, r1, r2, r3, r4, r5, r6, r7, r8, r9,
                g0, g1, g2, g3, g4, g5, g6, g7, g8, g9, acc_sh):
    rows = [r0, r1, r2, r3, r4, r5, r6, r7, r8, r9]
    gsems = [g0, g1, g2, g3, g4, g5, g6, g7, g8, g9]
    c = lax.axis_index("c")
    s = lax.axis_index("s")
    row0 = c * NP + s * STRIPE

    pltpu.sync_copy(hs_hbm.at[pl.ds(row0, STRIPE)],
                    acc_sh.at[pl.ds(s * STRIPE, STRIPE)])

    pltpu.sync_copy(srcs2_hbm.at[c * 16 + s], src_t)
    pltpu.sync_copy(dst2_hbm.at[s], dst_t)
    plsc.subcore_barrier()

    for b in range(NBUF):
        pltpu.async_copy(hs_hbm.at[src_t.at[b]], rows[b], gsems[b])

    def gbody(g, _):
        for b in range(NBUF):
            j = g * NBUF + b
            pltpu.make_async_copy(
                hs_hbm.at[src_t.at[j]], rows[b], gsems[b]).wait()
            pltpu.sync_copy(rows[b], acc_sh.at[dst_t.at[j]], add=True)

            @pl.when(j + NBUF < CHUNKS_T)
            def _():
                pltpu.async_copy(
                    hs_hbm.at[src_t.at[j + NBUF]], rows[b], gsems[b])
        return 0
    lax.fori_loop(0, MAIN_CHUNKS // NBUF, gbody, 0)

    for jj in range(MAIN_CHUNKS, CHUNKS_T):
        b = jj % NBUF
        pltpu.make_async_copy(
            hs_hbm.at[src_t.at[jj]], rows[b], gsems[b]).wait()
        pltpu.sync_copy(rows[b], acc_sh.at[dst_t.at[jj]], add=True)

    plsc.subcore_barrier()
    pltpu.sync_copy(acc_sh.at[pl.ds(s * STRIPE, STRIPE)],
                    out_hbm.at[pl.ds(row0, STRIPE)])


def _dinv_of(deg_ref):
    return lax.rsqrt(deg_ref[:, 0] + deg_ref[:, 1] + 1.0)


def _write_quarters(h, o0_ref, o1_ref):
    o0_ref[0] = h[:, 0 * QD:1 * QD]
    o0_ref[1] = h[:, 1 * QD:2 * QD]
    o1_ref[0] = h[:, 2 * QD:3 * QD]
    o1_ref[1] = h[:, 3 * QD:4 * QD]


def _cat_quarters(a00_ref, a01_ref, a10_ref, a11_ref):
    return jnp.concatenate(
        [a00_ref[0], a01_ref[0], a10_ref[0], a11_ref[0]], axis=1)


_qout_specs = [
    pl.BlockSpec((2, TN, QD), lambda i: (0, i, 0)),
    pl.BlockSpec((2, TN, QD), lambda i: (0, i, 0)),
]
_qout_shapes = [
    jax.ShapeDtypeStruct((2, NP, QD), jnp.float32),
    jax.ShapeDtypeStruct((2, NP, QD), jnp.float32),
]
_qin_specs = [
    pl.BlockSpec((1, TN, QD), lambda i: (0, i, 0)),
    pl.BlockSpec((1, TN, QD), lambda i: (1, i, 0)),
    pl.BlockSpec((1, TN, QD), lambda i: (0, i, 0)),
    pl.BlockSpec((1, TN, QD), lambda i: (1, i, 0)),
]


def _pre_kernel(x_ref, w_ref, deg_ref, o0_ref, o1_ref):
    dinv = _dinv_of(deg_ref)
    h = jax.lax.dot_general(
        x_ref[...], w_ref[...], (((1,), (0,)), ((), ())),
        preferred_element_type=jnp.float32)
    _write_quarters(h * dinv[:, None], o0_ref, o1_ref)


def _tc_pre(x, W1, deg2):
    return pl.pallas_call(
        _pre_kernel,
        grid=(GN,),
        in_specs=[
            pl.BlockSpec((TN, D), lambda i: (i, 0)),
            pl.BlockSpec((D, D), lambda i: (0, 0)),
            pl.BlockSpec((TN, 2), lambda i: (i, 0)),
        ],
        out_specs=_qout_specs,
        out_shape=_qout_shapes,
    )(x, W1, deg2)


def _mid_kernel(a00_ref, a01_ref, a10_ref, a11_ref, deg_ref, b_ref, w_ref,
                o0_ref, o1_ref):
    dinv = _dinv_of(deg_ref)
    acc = _cat_quarters(a00_ref, a01_ref, a10_ref, a11_ref)
    r = jnp.maximum(acc * dinv[:, None] + b_ref[...], 0.0)
    h = jax.lax.dot_general(
        r, w_ref[...], (((1,), (0,)), ((), ())),
        preferred_element_type=jnp.float32)
    _write_quarters(h * dinv[:, None], o0_ref, o1_ref)


def _tc_mid(acc1_p0, acc1_p1, deg2, b1, W2):
    return pl.pallas_call(
        _mid_kernel,
        grid=(GN,),
        in_specs=_qin_specs[:2] + _qin_specs[2:] + [
            pl.BlockSpec((TN, 2), lambda i: (i, 0)),
            pl.BlockSpec((1, D), lambda i: (0, 0)),
            pl.BlockSpec((D, D), lambda i: (0, 0)),
        ],
        out_specs=_qout_specs,
        out_shape=_qout_shapes,
    )(acc1_p0, acc1_p0, acc1_p1, acc1_p1, deg2, b1.reshape(1, D), W2)


def _tail_kernel(a00_ref, a01_ref, a10_ref, a11_ref, deg_ref, b_ref,
                 w1s_ref, b1s_ref, w2s_ref, b2s_ref, o_ref):
    dinv = _dinv_of(deg_ref)
    acc = _cat_quarters(a00_ref, a01_ref, a10_ref, a11_ref)
    h = acc * dinv[:, None] + b_ref[...]
    n1 = w1s_ref.shape[0]
    n2 = w2s_ref.shape[0]
    for i in range(n1):
        h = jax.lax.dot_general(
            h, w1s_ref[i], (((1,), (0,)), ((), ())),
            preferred_element_type=jnp.float32) + b1s_ref[i]
        h = jnp.maximum(h, 0.0)
    for i in range(n2):
        h = jax.lax.dot_general(
            h, w2s_ref[i], (((1,), (0,)), ((), ())),
            preferred_element_type=jnp.float32) + b2s_ref[i]
        if i != n2 - 1:
            h = jnp.maximum(h, 0.0)
    o_ref[...] = h


def _tc_tail(acc2_p0, acc2_p1, deg2, b2, W1s, b1s, W2s, b2s):
    L = W1s.shape[0]
    return pl.pallas_call(
        _tail_kernel,
        grid=(GN,),
        in_specs=_qin_specs[:2] + _qin_specs[2:] + [
            pl.BlockSpec((TN, 2), lambda i: (i, 0)),
            pl.BlockSpec((1, D), lambda i: (0, 0)),
            pl.BlockSpec((L, D, D), lambda i: (0, 0, 0)),
            pl.BlockSpec((L, 1, D), lambda i: (0, 0, 0)),
            pl.BlockSpec((L, D, D), lambda i: (0, 0, 0)),
            pl.BlockSpec((L, 1, D), lambda i: (0, 0, 0)),
        ],
        out_specs=pl.BlockSpec((TN, D), lambda i: (i, 0)),
        out_shape=jax.ShapeDtypeStruct((N, D), jnp.float32),
    )(acc2_p0, acc2_p0, acc2_p1, acc2_p1, deg2, b2.reshape(1, D),
      W1s, b1s.reshape(L, 1, D), W2s, b2s.reshape(L, 1, D))


def kernel(x, edge_index, batch, W1, b1, W2, b2, mlp1_W, mlp1_b, mlp2_W, mlp2_b):
    x = x.astype(jnp.float32)
    src = edge_index[0].astype(jnp.int32)
    dst = edge_index[1].astype(jnp.int32)

    srcs2 = jnp.concatenate([src, src + NP]).reshape(32, CHUNKS_T, K)
    dst2 = dst.reshape(16, CHUNKS_T, K)

    deg2 = _deg_kernel(dst).T

    hs1_p0, hs1_p1 = _tc_pre(x, W1, deg2)
    acc1_p0 = _agg_kernel(hs1_p0.reshape(2 * NP, QD), srcs2, dst2)
    acc1_p1 = _agg_kernel(hs1_p1.reshape(2 * NP, QD), srcs2, dst2)
    acc1_p0 = acc1_p0.reshape(2, NP, QD)
    acc1_p1 = acc1_p1.reshape(2, NP, QD)

    hs2_p0, hs2_p1 = _tc_mid(acc1_p0, acc1_p1, deg2, b1, W2)
    acc2_p0 = _agg_kernel(hs2_p0.reshape(2 * NP, QD), srcs2, dst2)
    acc2_p1 = _agg_kernel(hs2_p1.reshape(2 * NP, QD), srcs2, dst2)
    acc2_p0 = acc2_p0.reshape(2, NP, QD)
    acc2_p1 = acc2_p1.reshape(2, NP, QD)

    return _tc_tail(acc2_p0, acc2_p1, deg2, b2,
                    mlp1_W, mlp1_b, mlp2_W, mlp2_b)


# speedup vs baseline: 1.0166x; 1.0166x over previous
import functools
import jax
import jax.numpy as jnp
from jax import lax
from jax.experimental import pallas as pl
from jax.experimental.pallas import tpu as pltpu
from jax.experimental.pallas import tpu_sc as plsc

N = 10000
NP = 10240
D = 256
QD = 64
E = 160000
TN = 2000
GN = N // TN

K = 80
NBUF = 5
EC_T = E // 16
CHUNKS_T = EC_T // K

ET = E // 32
NV = ET // 16
STRIPE = NP // 16

_mesh = plsc.VectorSubcoreMesh(core_axis_name="c", subcore_axis_name="s")
_sc_params = pltpu.CompilerParams(needs_layout_passes=False)
_sc_agg_params = pltpu.CompilerParams(
    needs_layout_passes=False, use_tc_tiling_on_sc=False)


@functools.partial(
    pl.kernel, mesh=_mesh, compiler_params=_sc_params,
    out_type=jax.ShapeDtypeStruct((2, NP), jnp.float32),
    scratch_types=[
        pltpu.VMEM((ET,), jnp.int32),
        pltpu.VMEM((NP,), jnp.float32),
        pltpu.VMEM((STRIPE,), jnp.float32),
        pltpu.VMEM((STRIPE,), jnp.float32),
        pltpu.VMEM_SHARED((16, NP), jnp.float32),
    ])
def _deg_kernel(dst_hbm, deg_out, dst_v, acc_v, red_v, tmp_v, part_sh):
    c = lax.axis_index("c")
    s = lax.axis_index("s")
    tid = c * 16 + s
    pltpu.sync_copy(dst_hbm.at[pl.ds(tid * ET, ET)], dst_v)

    zero16 = jnp.zeros((16,), jnp.float32)
    ones16 = jnp.ones((16,), jnp.float32)

    def zbody(i, _):
        acc_v[pl.ds(i * 16, 16)] = zero16
        return 0
    lax.fori_loop(0, NP // 16, zbody, 0)

    def body(i, _):
        idx = dst_v[pl.ds(i * 16, 16)]
        plsc.addupdate_scatter(acc_v, [idx], ones16)
        return 0
    lax.fori_loop(0, NV, body, 0)
    rem_idx = dst_v[pl.ds(ET - 16, 16)]
    rem_mask = lax.iota(jnp.int32, 16) >= (16 - (ET - NV * 16))
    plsc.addupdate_scatter(acc_v, [rem_idx], ones16, mask=rem_mask)

    pltpu.sync_copy(acc_v, part_sh.at[s])
    plsc.subcore_barrier()

    sl = pl.ds(s * STRIPE, STRIPE)
    pltpu.sync_copy(part_sh.at[0, sl], red_v)

    def rbody(j, _):
        pltpu.sync_copy(part_sh.at[j, sl], tmp_v)

        def abody(i, _):
            w = pl.ds(i * 16, 16)
            red_v[w] = red_v[w] + tmp_v[w]
            return 0
        lax.fori_loop(0, STRIPE // 16, abody, 0)
        return 0
    lax.fori_loop(1, 16, rbody, 0)
    pltpu.sync_copy(red_v, deg_out.at[c, sl])


@functools.partial(
    pl.kernel, mesh=_mesh, compiler_params=_sc_agg_params,
    out_type=jax.ShapeDtypeStruct((2 * NP, QD), jnp.float32),
    scratch_types=[
        pltpu.VMEM((CHUNKS_T, K), jnp.int32),
        pltpu.VMEM((CHUNKS_T, K), jnp.int32),
    ] + [pltpu.VMEM((K, QD), jnp.float32) for _ in range(NBUF)]
      + [pltpu.SemaphoreType.DMA for _ in range(NBUF)]
      + [pltpu.VMEM_SHARED((NP, QD), jnp.float32)])
def _agg_kernel(hs_hbm, srcs2_hbm, dst2_hbm, out_hbm, src_t, dst_t,
                r0, r1, r2, r3, r4, g0, g1, g2, g3, g4, acc_sh):
    rows = [r0, r1, r2, r3, r4]
    gsems = [g0, g1, g2, g3, g4]
    c = lax.axis_index("c")
    s = lax.axis_index("s")
    row0 = c * NP + s * STRIPE

    pltpu.sync_copy(hs_hbm.at[pl.ds(row0, STRIPE)],
                    acc_sh.at[pl.ds(s * STRIPE, STRIPE)])

    pltpu.sync_copy(srcs2_hbm.at[c * 16 + s], src_t)
    pltpu.sync_copy(dst2_hbm.at[s], dst_t)
    plsc.subcore_barrier()

    for b in range(NBUF):
        pltpu.async_copy(hs_hbm.at[src_t.at[b]], rows[b], gsems[b])

    def gbody(g, _):
        for b in range(NBUF):
            j = g * NBUF + b
            pltpu.make_async_copy(
                hs_hbm.at[src_t.at[j]], rows[b], gsems[b]).wait()
            pltpu.sync_copy(rows[b], acc_sh.at[dst_t.at[j]], add=True)

            @pl.when(j + NBUF < CHUNKS_T)
            def _():
                pltpu.async_copy(
                    hs_hbm.at[src_t.at[j + NBUF]], rows[b], gsems[b])
        return 0
    lax.fori_loop(0, CHUNKS_T // NBUF, gbody, 0)

    plsc.subcore_barrier()
    pltpu.sync_copy(acc_sh.at[pl.ds(s * STRIPE, STRIPE)],
                    out_hbm.at[pl.ds(row0, STRIPE)])


def _dinv_of(deg_ref):
    return lax.rsqrt(deg_ref[:, 0] + deg_ref[:, 1] + 1.0)


def _write_quarters(h, o0_ref, o1_ref):
    o0_ref[0] = h[:, 0 * QD:1 * QD]
    o0_ref[1] = h[:, 1 * QD:2 * QD]
    o1_ref[0] = h[:, 2 * QD:3 * QD]
    o1_ref[1] = h[:, 3 * QD:4 * QD]


def _cat_quarters(a00_ref, a01_ref, a10_ref, a11_ref):
    return jnp.concatenate(
        [a00_ref[0], a01_ref[0], a10_ref[0], a11_ref[0]], axis=1)


_qout_specs = [
    pl.BlockSpec((2, TN, QD), lambda i: (0, i, 0)),
    pl.BlockSpec((2, TN, QD), lambda i: (0, i, 0)),
]
_qout_shapes = [
    jax.ShapeDtypeStruct((2, NP, QD), jnp.float32),
    jax.ShapeDtypeStruct((2, NP, QD), jnp.float32),
]
_qin_specs = [
    pl.BlockSpec((1, TN, QD), lambda i: (0, i, 0)),
    pl.BlockSpec((1, TN, QD), lambda i: (1, i, 0)),
    pl.BlockSpec((1, TN, QD), lambda i: (0, i, 0)),
    pl.BlockSpec((1, TN, QD), lambda i: (1, i, 0)),
]


def _pre_kernel(x_ref, w_ref, deg_ref, o0_ref, o1_ref):
    dinv = _dinv_of(deg_ref)
    h = jax.lax.dot_general(
        x_ref[...], w_ref[...], (((1,), (0,)), ((), ())),
        preferred_element_type=jnp.float32)
    _write_quarters(h * dinv[:, None], o0_ref, o1_ref)


def _tc_pre(x, W1, deg2):
    return pl.pallas_call(
        _pre_kernel,
        grid=(GN,),
        in_specs=[
            pl.BlockSpec((TN, D), lambda i: (i, 0)),
            pl.BlockSpec((D, D), lambda i: (0, 0)),
            pl.BlockSpec((TN, 2), lambda i: (i, 0)),
        ],
        out_specs=_qout_specs,
        out_shape=_qout_shapes,
    )(x, W1, deg2)


def _mid_kernel(a00_ref, a01_ref, a10_ref, a11_ref, deg_ref, b_ref, w_ref,
                o0_ref, o1_ref):
    dinv = _dinv_of(deg_ref)
    acc = _cat_quarters(a00_ref, a01_ref, a10_ref, a11_ref)
    r = jnp.maximum(acc * dinv[:, None] + b_ref[...], 0.0)
    h = jax.lax.dot_general(
        r, w_ref[...], (((1,), (0,)), ((), ())),
        preferred_element_type=jnp.float32)
    _write_quarters(h * dinv[:, None], o0_ref, o1_ref)


def _tc_mid(acc1_p0, acc1_p1, deg2, b1, W2):
    return pl.pallas_call(
        _mid_kernel,
        grid=(GN,),
        in_specs=_qin_specs[:2] + _qin_specs[2:] + [
            pl.BlockSpec((TN, 2), lambda i: (i, 0)),
            pl.BlockSpec((1, D), lambda i: (0, 0)),
            pl.BlockSpec((D, D), lambda i: (0, 0)),
        ],
        out_specs=_qout_specs,
        out_shape=_qout_shapes,
    )(acc1_p0, acc1_p0, acc1_p1, acc1_p1, deg2, b1.reshape(1, D), W2)


def _tail_kernel(a00_ref, a01_ref, a10_ref, a11_ref, deg_ref, b_ref,
                 w1s_ref, b1s_ref, w2s_ref, b2s_ref, o_ref):
    dinv = _dinv_of(deg_ref)
    acc = _cat_quarters(a00_ref, a01_ref, a10_ref, a11_ref)
    h = acc * dinv[:, None] + b_ref[...]
    n1 = w1s_ref.shape[0]
    n2 = w2s_ref.shape[0]
    for i in range(n1):
        h = jax.lax.dot_general(
            h, w1s_ref[i], (((1,), (0,)), ((), ())),
            preferred_element_type=jnp.float32) + b1s_ref[i]
        h = jnp.maximum(h, 0.0)
    for i in range(n2):
        h = jax.lax.dot_general(
            h, w2s_ref[i], (((1,), (0,)), ((), ())),
            preferred_element_type=jnp.float32) + b2s_ref[i]
        if i != n2 - 1:
            h = jnp.maximum(h, 0.0)
    o_ref[...] = h


def _tc_tail(acc2_p0, acc2_p1, deg2, b2, W1s, b1s, W2s, b2s):
    L = W1s.shape[0]
    return pl.pallas_call(
        _tail_kernel,
        grid=(GN,),
        in_specs=_qin_specs[:2] + _qin_specs[2:] + [
            pl.BlockSpec((TN, 2), lambda i: (i, 0)),
            pl.BlockSpec((1, D), lambda i: (0, 0)),
            pl.BlockSpec((L, D, D), lambda i: (0, 0, 0)),
            pl.BlockSpec((L, 1, D), lambda i: (0, 0, 0)),
            pl.BlockSpec((L, D, D), lambda i: (0, 0, 0)),
            pl.BlockSpec((L, 1, D), lambda i: (0, 0, 0)),
        ],
        out_specs=pl.BlockSpec((TN, D), lambda i: (i, 0)),
        out_shape=jax.ShapeDtypeStruct((N, D), jnp.float32),
    )(acc2_p0, acc2_p0, acc2_p1, acc2_p1, deg2, b2.reshape(1, D),
      W1s, b1s.reshape(L, 1, D), W2s, b2s.reshape(L, 1, D))


def kernel(x, edge_index, batch, W1, b1, W2, b2, mlp1_W, mlp1_b, mlp2_W, mlp2_b):
    x = x.astype(jnp.float32)
    src = edge_index[0].astype(jnp.int32)
    dst = edge_index[1].astype(jnp.int32)

    srcs2 = jnp.concatenate([src, src + NP]).reshape(32, CHUNKS_T, K)
    dst2 = dst.reshape(16, CHUNKS_T, K)

    deg2 = _deg_kernel(dst).T

    hs1_p0, hs1_p1 = _tc_pre(x, W1, deg2)
    acc1_p0 = _agg_kernel(hs1_p0.reshape(2 * NP, QD), srcs2, dst2)
    acc1_p1 = _agg_kernel(hs1_p1.reshape(2 * NP, QD), srcs2, dst2)
    acc1_p0 = acc1_p0.reshape(2, NP, QD)
    acc1_p1 = acc1_p1.reshape(2, NP, QD)

    hs2_p0, hs2_p1 = _tc_mid(acc1_p0, acc1_p1, deg2, b1, W2)
    acc2_p0 = _agg_kernel(hs2_p0.reshape(2 * NP, QD), srcs2, dst2)
    acc2_p1 = _agg_kernel(hs2_p1.reshape(2 * NP, QD), srcs2, dst2)
    acc2_p0 = acc2_p0.reshape(2, NP, QD)
    acc2_p1 = acc2_p1.reshape(2, NP, QD)

    return _tc_tail(acc2_p0, acc2_p1, deg2, b2,
                    mlp1_W, mlp1_b, mlp2_W, mlp2_b)
